# baseline (speedup 1.0000x reference)
import dataclasses
import functools

import jax
import jax.numpy as jnp
from jax import lax
from jax.experimental import pallas as pl
from jax.experimental.pallas import tpu as pltpu
from jax.experimental.pallas import tpu_sc as plsc

N = 10000
E = 160000
D = 256

NC = 2
NS = 16
NW = NC * NS
RPT = 320
NPAD = NW * RPT
CHUNK = 1600
NCHUNK = E // CHUNK
LANES = 16
GRP = CHUNK // LANES
GBLK = 128
QCAP = CHUNK + GBLK + 48
THRESH = QCAP - CHUNK - 32
DV = D // LANES


def _mm_body(x_ref, w_ref, o_ref):
    o_ref[...] = jnp.dot(x_ref[...], w_ref[...],
                         preferred_element_type=jnp.float32)


def _matmul(x, W):
    BM = 400
    return pl.pallas_call(
        _mm_body,
        grid=(N // BM,),
        in_specs=[
            pl.BlockSpec((BM, D), lambda i: (i, 0)),
            pl.BlockSpec((D, D), lambda i: (0, 0)),
        ],
        out_specs=pl.BlockSpec((BM, D), lambda i: (i, 0)),
        out_shape=jax.ShapeDtypeStruct((N, D), jnp.float32),
    )(x, W)


def _sc_aggregate(tmp, src, dst, w):
    mesh = plsc.VectorSubcoreMesh(core_axis_name="c", subcore_axis_name="s")
    cp = pltpu.CompilerParams()
    if "needs_layout_passes" in pltpu.CompilerParams.__dataclass_fields__:
        cp = dataclasses.replace(cp, needs_layout_passes=False)
    if "use_tc_tiling_on_sc" in pltpu.CompilerParams.__dataclass_fields__:
        cp = dataclasses.replace(cp, use_tc_tiling_on_sc=False)

    @functools.partial(
        pl.kernel,
        out_type=jax.ShapeDtypeStruct((NPAD, D), jnp.float32),
        mesh=mesh,
        compiler_params=cp,
        scratch_types=[
            pltpu.VMEM((RPT, D), jnp.float32),
            pltpu.VMEM((GBLK, D), jnp.float32),
            pltpu.VMEM((CHUNK,), jnp.int32),
            pltpu.VMEM((CHUNK,), jnp.int32),
            pltpu.VMEM((CHUNK,), jnp.float32),
            pltpu.VMEM((CHUNK,), jnp.int32),
            pltpu.VMEM((CHUNK,), jnp.int32),
            pltpu.VMEM((CHUNK,), jnp.float32),
            pltpu.VMEM((QCAP,), jnp.int32),
            pltpu.VMEM((QCAP,), jnp.int32),
            pltpu.VMEM((QCAP,), jnp.float32),
            pltpu.SemaphoreType.DMA,
            pltpu.SemaphoreType.DMA,
            pltpu.SemaphoreType.DMA,
        ],
    )
    def agg(tmp_hbm, src_hbm, dst_hbm, w_hbm, out_hbm,
            acc, rows, c0s, c0d, c0w, c1s, c1d, c1w,
            qsrc, qdl, qw, semA, semB, gsem):
        wid = lax.axis_index("s") * NC + lax.axis_index("c")
        lo = wid * RPT

        zi = jnp.zeros((LANES,), jnp.int32)
        zf = jnp.zeros((LANES,), jnp.float32)
        lane = lax.iota(jnp.int32, LANES)

        @pl.loop(0, RPT)
        def _(r):
            for j in range(DV):
                acc[r, pl.ds(j * LANES, LANES)] = zf

        @pl.loop(0, QCAP // LANES)
        def _(i):
            sl = pl.ds(i * LANES, LANES)
            qsrc[sl] = zi
            qdl[sl] = zi
            qw[sl] = zf

        def chunk_copies(c, bufs, sem):
            base = c * CHUNK
            s, d, wv = bufs
            return (
                pltpu.make_async_copy(src_hbm.at[pl.ds(base, CHUNK)], s, sem),
                pltpu.make_async_copy(dst_hbm.at[pl.ds(base, CHUNK)], d, sem),
                pltpu.make_async_copy(w_hbm.at[pl.ds(base, CHUNK)], wv, sem),
            )

        def gather_copy():
            return pltpu.make_async_copy(
                tmp_hbm.at[qsrc.at[pl.ds(0, GBLK)]], rows, gsem)

        def scan_chunk(bufs, qn):
            s, d, wv = bufs

            SUNROLL = 4

            def scan_body(gb, qn):
                g = gb * SUNROLL
                sls = [pl.ds((g + u) * LANES, LANES) for u in range(SUNROLL)]
                dvs = [d[sl] for sl in sls]
                svs = [s[sl] for sl in sls]
                wvs = [wv[sl] for sl in sls]
                ms = [(dv >= lo) & (dv < lo + RPT) for dv in dvs]
                cs = [plsc.all_reduce_population_count(m)[0] for m in ms]
                for u in range(SUNROLL):
                    qsl = pl.ds(qn, LANES)
                    plsc.store_compressed(qdl.at[qsl], dvs[u] - lo,
                                          mask=ms[u])
                    plsc.store_compressed(qsrc.at[qsl], svs[u], mask=ms[u])
                    plsc.store_compressed(qw.at[qsl], wvs[u], mask=ms[u])
                    qn = qn + cs[u]
                return qn

            return lax.fori_loop(0, GRP // SUNROLL, scan_body, qn)

        def do_accum(n_valid):
            @pl.loop(0, GBLK // LANES)
            def _(g):
                qoff = g * LANES
                wv = qw[pl.ds(qoff, LANES)]
                dl = qdl[pl.ds(qoff, LANES)]
                valid = (qoff + lane) < n_valid
                wv = jnp.where(valid, wv, 0.0)
                def row_vals(kk):
                    wk = wv[kk]
                    rr = qoff + kk
                    return [rows[rr, pl.ds(j * LANES, LANES)] * wk
                            for j in range(DV)]

                vals = row_vals(0)
                for kk in range(LANES):
                    row = dl[kk]
                    if kk + 1 < LANES:
                        wk1 = wv[kk + 1]
                        rr1 = qoff + kk + 1
                        nxt = []
                        for j in range(DV):
                            sl = pl.ds(j * LANES, LANES)
                            nxt.append(rows[rr1, sl] * wk1)
                            plsc.addupdate(acc.at[row, sl], vals[j])
                        vals = nxt
                    else:
                        for j in range(DV):
                            sl = pl.ds(j * LANES, LANES)
                            plsc.addupdate(acc.at[row, sl], vals[j])

        def shift_queue(qn):
            n16 = (qn - GBLK + LANES - 1) // LANES

            def mv(k, carry):
                s_sl = pl.ds(GBLK + k * LANES, LANES)
                d_sl = pl.ds(k * LANES, LANES)
                qsrc[d_sl] = qsrc[s_sl]
                qdl[d_sl] = qdl[s_sl]
                qw[d_sl] = qw[s_sl]
                return carry

            lax.fori_loop(0, n16, mv, jnp.int32(0))

        def retire_async(qn):
            gather_copy().wait()
            do_accum(jnp.int32(GBLK))
            shift_queue(qn)
            return qn - GBLK

        def sync_retire(qn):
            pltpu.sync_copy(tmp_hbm.at[qsrc.at[pl.ds(0, GBLK)]], rows)
            do_accum(jnp.int32(GBLK))
            shift_queue(qn)
            return qn - GBLK

        def process_chunk(cidx, my_bufs, my_sem, other_bufs, other_sem,
                          qn, pend):
            @pl.when(cidx + 1 < NCHUNK)
            def _():
                for cp_ in chunk_copies(cidx + 1, other_bufs, other_sem):
                    cp_.start()

            for cp_ in chunk_copies(cidx, my_bufs, my_sem):
                cp_.wait()

            qn = scan_chunk(my_bufs, qn)
            qn = lax.cond(pend == 1, retire_async, lambda q: q, qn)
            qn = lax.while_loop(lambda q: q >= THRESH, sync_retire, qn)
            pend = (qn >= GBLK).astype(jnp.int32)

            @pl.when(pend == 1)
            def _():
                gather_copy().start()

            return qn, pend

        set0 = (c0s, c0d, c0w)
        set1 = (c1s, c1d, c1w)

        for cp_ in chunk_copies(jnp.int32(0), set0, semA):
            cp_.start()

        def outer(i, carry):
            qn, pend = carry
            qn, pend = process_chunk(2 * i, set0, semA, set1, semB, qn, pend)
            qn, pend = process_chunk(2 * i + 1, set1, semB, set0, semA,
                                     qn, pend)
            return qn, pend

        qn, pend = lax.fori_loop(0, NCHUNK // 2, outer,
                                 (jnp.int32(0), jnp.int32(0)))

        qn = lax.cond(pend == 1, retire_async, lambda q: q, qn)
        qn = lax.while_loop(lambda q: q >= GBLK, sync_retire, qn)

        @pl.when(qn > 0)
        def _():
            pltpu.sync_copy(tmp_hbm.at[qsrc.at[pl.ds(0, GBLK)]], rows)
            do_accum(qn)

        @pl.loop(0, RPT)
        def _(r):
            vals = [jnp.maximum(acc[r, pl.ds(j * LANES, LANES)], 0.0)
                    for j in range(DV)]
            for j in range(DV):
                acc[r, pl.ds(j * LANES, LANES)] = vals[j]

        pltpu.sync_copy(acc, out_hbm.at[pl.ds(lo, RPT)])

    return agg(tmp, src, dst, w)


@jax.jit
def kernel(x, edge_index, edge_weight, W):
    tmp = _matmul(x, W)
    src = edge_index[0]
    dst = edge_index[1]
    out = _sc_aggregate(tmp, src, dst, edge_weight)
    return out[:N]


# speedup vs baseline: 1.0506x; 1.0506x over previous
import dataclasses
import functools

import jax
import jax.numpy as jnp
from jax import lax
from jax.experimental import pallas as pl
from jax.experimental.pallas import tpu as pltpu
from jax.experimental.pallas import tpu_sc as plsc

N = 10000
E = 160000
D = 256

NC = 2
NS = 16
NW = NC * NS
RPT = 320
NPAD = NW * RPT
CHUNK = 1600
NCHUNK = E // CHUNK
LANES = 16
GRP = CHUNK // LANES
GBLK = 128
QCAP = CHUNK + GBLK + 48
THRESH = QCAP - CHUNK - 32
DV = D // LANES


def _mm_body(x_ref, w_ref, o_ref):
    o_ref[...] = jnp.dot(x_ref[...], w_ref[...],
                         preferred_element_type=jnp.float32)


def _matmul(x, W):
    BM = 400
    return pl.pallas_call(
        _mm_body,
        grid=(N // BM,),
        in_specs=[
            pl.BlockSpec((BM, D), lambda i: (i, 0)),
            pl.BlockSpec((D, D), lambda i: (0, 0)),
        ],
        out_specs=pl.BlockSpec((BM, D), lambda i: (i, 0)),
        out_shape=jax.ShapeDtypeStruct((N, D), jnp.float32),
    )(x, W)


def _sc_aggregate(tmp, src, dst, w):
    mesh = plsc.VectorSubcoreMesh(core_axis_name="c", subcore_axis_name="s")
    cp = pltpu.CompilerParams()
    if "needs_layout_passes" in pltpu.CompilerParams.__dataclass_fields__:
        cp = dataclasses.replace(cp, needs_layout_passes=False)

    @functools.partial(
        pl.kernel,
        out_type=jax.ShapeDtypeStruct((NPAD, D), jnp.float32),
        mesh=mesh,
        compiler_params=cp,
        scratch_types=[
            pltpu.VMEM((RPT, D), jnp.float32),
            pltpu.VMEM((GBLK, D), jnp.float32),
            pltpu.VMEM((CHUNK,), jnp.int32),
            pltpu.VMEM((CHUNK,), jnp.int32),
            pltpu.VMEM((CHUNK,), jnp.float32),
            pltpu.VMEM((CHUNK,), jnp.int32),
            pltpu.VMEM((CHUNK,), jnp.int32),
            pltpu.VMEM((CHUNK,), jnp.float32),
            pltpu.VMEM((QCAP,), jnp.int32),
            pltpu.VMEM((QCAP,), jnp.int32),
            pltpu.VMEM((QCAP,), jnp.float32),
            pltpu.SemaphoreType.DMA,
            pltpu.SemaphoreType.DMA,
            pltpu.SemaphoreType.DMA,
        ],
    )
    def agg(tmp_hbm, src_hbm, dst_hbm, w_hbm, out_hbm,
            acc, rows, c0s, c0d, c0w, c1s, c1d, c1w,
            qsrc, qdl, qw, semA, semB, gsem):
        wid = lax.axis_index("s") * NC + lax.axis_index("c")
        lo = wid * RPT

        zi = jnp.zeros((LANES,), jnp.int32)
        zf = jnp.zeros((LANES,), jnp.float32)
        lane = lax.iota(jnp.int32, LANES)

        @pl.loop(0, RPT)
        def _(r):
            for j in range(DV):
                acc[r, pl.ds(j * LANES, LANES)] = zf

        @pl.loop(0, QCAP // LANES)
        def _(i):
            sl = pl.ds(i * LANES, LANES)
            qsrc[sl] = zi
            qdl[sl] = zi
            qw[sl] = zf

        def chunk_copies(c, bufs, sem):
            base = c * CHUNK
            s, d, wv = bufs
            return (
                pltpu.make_async_copy(src_hbm.at[pl.ds(base, CHUNK)], s, sem),
                pltpu.make_async_copy(dst_hbm.at[pl.ds(base, CHUNK)], d, sem),
                pltpu.make_async_copy(w_hbm.at[pl.ds(base, CHUNK)], wv, sem),
            )

        def gather_copy():
            return pltpu.make_async_copy(
                tmp_hbm.at[qsrc.at[pl.ds(0, GBLK)]], rows, gsem)

        def scan_chunk(bufs, qn):
            s, d, wv = bufs

            SUNROLL = 4

            def scan_body(gb, qn):
                g = gb * SUNROLL
                sls = [pl.ds((g + u) * LANES, LANES) for u in range(SUNROLL)]
                dvs = [d[sl] for sl in sls]
                svs = [s[sl] for sl in sls]
                wvs = [wv[sl] for sl in sls]
                ms = [(dv >= lo) & (dv < lo + RPT) for dv in dvs]
                cs = [plsc.all_reduce_population_count(m)[0] for m in ms]
                for u in range(SUNROLL):
                    qsl = pl.ds(qn, LANES)
                    plsc.store_compressed(qdl.at[qsl], dvs[u] - lo,
                                          mask=ms[u])
                    plsc.store_compressed(qsrc.at[qsl], svs[u], mask=ms[u])
                    plsc.store_compressed(qw.at[qsl], wvs[u], mask=ms[u])
                    qn = qn + cs[u]
                return qn

            return lax.fori_loop(0, GRP // SUNROLL, scan_body, qn)

        def do_accum(n_valid):
            @pl.loop(0, GBLK // LANES)
            def _(g):
                qoff = g * LANES
                wv = qw[pl.ds(qoff, LANES)]
                dl = qdl[pl.ds(qoff, LANES)]
                valid = (qoff + lane) < n_valid
                wv = jnp.where(valid, wv, 0.0)
                def row_vals(kk):
                    wk = wv[kk]
                    rr = qoff + kk
                    return [rows[rr, pl.ds(j * LANES, LANES)] * wk
                            for j in range(DV)]

                vals = row_vals(0)
                for kk in range(LANES):
                    row = dl[kk]
                    if kk + 1 < LANES:
                        wk1 = wv[kk + 1]
                        rr1 = qoff + kk + 1
                        nxt = []
                        for j in range(DV):
                            sl = pl.ds(j * LANES, LANES)
                            nxt.append(rows[rr1, sl] * wk1)
                            plsc.addupdate(acc.at[row, sl], vals[j])
                        vals = nxt
                    else:
                        for j in range(DV):
                            sl = pl.ds(j * LANES, LANES)
                            plsc.addupdate(acc.at[row, sl], vals[j])

        def shift_queue(qn):
            n16 = (qn - GBLK + LANES - 1) // LANES

            def mv(k, carry):
                s_sl = pl.ds(GBLK + k * LANES, LANES)
                d_sl = pl.ds(k * LANES, LANES)
                qsrc[d_sl] = qsrc[s_sl]
                qdl[d_sl] = qdl[s_sl]
                qw[d_sl] = qw[s_sl]
                return carry

            lax.fori_loop(0, n16, mv, jnp.int32(0))

        def retire_async(qn):
            gather_copy().wait()
            do_accum(jnp.int32(GBLK))
            shift_queue(qn)
            return qn - GBLK

        def sync_retire(qn):
            pltpu.sync_copy(tmp_hbm.at[qsrc.at[pl.ds(0, GBLK)]], rows)
            do_accum(jnp.int32(GBLK))
            shift_queue(qn)
            return qn - GBLK

        def process_chunk(cidx, my_bufs, my_sem, other_bufs, other_sem,
                          qn, pend):
            @pl.when(cidx + 1 < NCHUNK)
            def _():
                for cp_ in chunk_copies(cidx + 1, other_bufs, other_sem):
                    cp_.start()

            for cp_ in chunk_copies(cidx, my_bufs, my_sem):
                cp_.wait()

            qn = scan_chunk(my_bufs, qn)
            qn = lax.cond(pend == 1, retire_async, lambda q: q, qn)
            qn = lax.while_loop(lambda q: q >= THRESH, sync_retire, qn)
            pend = (qn >= GBLK).astype(jnp.int32)

            @pl.when(pend == 1)
            def _():
                gather_copy().start()

            return qn, pend

        set0 = (c0s, c0d, c0w)
        set1 = (c1s, c1d, c1w)

        for cp_ in chunk_copies(jnp.int32(0), set0, semA):
            cp_.start()

        def outer(i, carry):
            qn, pend = carry
            qn, pend = process_chunk(2 * i, set0, semA, set1, semB, qn, pend)
            qn, pend = process_chunk(2 * i + 1, set1, semB, set0, semA,
                                     qn, pend)
            return qn, pend

        qn, pend = lax.fori_loop(0, NCHUNK // 2, outer,
                                 (jnp.int32(0), jnp.int32(0)))

        qn = lax.cond(pend == 1, retire_async, lambda q: q, qn)
        qn = lax.while_loop(lambda q: q >= GBLK, sync_retire, qn)

        @pl.when(qn > 0)
        def _():
            pltpu.sync_copy(tmp_hbm.at[qsrc.at[pl.ds(0, GBLK)]], rows)
            do_accum(qn)

        @pl.loop(0, RPT)
        def _(r):
            vals = [jnp.maximum(acc[r, pl.ds(j * LANES, LANES)], 0.0)
                    for j in range(DV)]
            for j in range(DV):
                acc[r, pl.ds(j * LANES, LANES)] = vals[j]

        pltpu.sync_copy(acc, out_hbm.at[pl.ds(lo, RPT)])

    return agg(tmp, src, dst, w)


@jax.jit
def kernel(x, edge_index, edge_weight, W):
    tmp = _matmul(x, W)
    src = edge_index[0]
    dst = edge_index[1]
    out = _sc_aggregate(tmp, src, dst, edge_weight)
    return out[:N]
